# Initial kernel scaffold; baseline (speedup 1.0000x reference)
import jax
import jax.numpy as jnp
from jax.experimental import pallas as pl


def kernel(loc_data, conf_data, priors, targets):
    raise NotImplementedError("write your pallas kernel here")


# speedup vs baseline: 7.0559x; 7.0559x over previous
import jax
import jax.numpy as jnp
from jax import lax
from jax.experimental import pallas as pl
from jax.experimental.pallas import tpu as pltpu

_THRESHOLD = 0.35
_V0, _V1 = 0.1, 0.2
_NEG_RATIO = 3
_T = 16


def _row_kernel(pcx_ref, pcy_ref, pw_ref, ph_ref,
                tx0_ref, ty0_ref, tx1_ref, ty1_ref,
                l0_ref, l1_ref, l2_ref, l3_ref,
                c0_ref, c1_ref,
                out_ref, acc_ref):
    b = pl.program_id(0)
    nb = pl.num_programs(0)
    P = pcx_ref.shape[1]

    pcx = pcx_ref[...]
    pcy = pcy_ref[...]
    pw = pw_ref[...]
    ph = ph_ref[...]
    px0 = pcx - pw / 2.0
    py0 = pcy - ph / 2.0
    px1 = pcx + pw / 2.0
    py1 = pcy + ph / 2.0
    area_p = (px1 - px0) * (py1 - py0)

    iota = lax.broadcasted_iota(jnp.int32, (1, P), 1)

    bv = jnp.full((1, P), -1.0, dtype=jnp.float32)
    tm_x0 = jnp.zeros((1, P), dtype=jnp.float32)
    tm_y0 = jnp.zeros((1, P), dtype=jnp.float32)
    tm_x1 = jnp.zeros((1, P), dtype=jnp.float32)
    tm_y1 = jnp.zeros((1, P), dtype=jnp.float32)
    bp_idx = []
    truth_coords = []

    for t in range(_T):
        tx0 = tx0_ref[b, t]
        ty0 = ty0_ref[b, t]
        tx1 = tx1_ref[b, t]
        ty1 = ty1_ref[b, t]
        truth_coords.append((tx0, ty0, tx1, ty1))
        iw = jnp.clip(jnp.minimum(tx1, px1) - jnp.maximum(tx0, px0), 0.0, None)
        ih = jnp.clip(jnp.minimum(ty1, py1) - jnp.maximum(ty0, py0), 0.0, None)
        inter = iw * ih
        area_t = (tx1 - tx0) * (ty1 - ty0)
        union = area_t + area_p - inter
        iou = inter / union
        m_t = jnp.max(iou)
        idx_t = jnp.min(jnp.where(iou == m_t, iota, P))
        bp_idx.append(idx_t)
        upd = iou > bv
        bv = jnp.where(upd, iou, bv)
        tm_x0 = jnp.where(upd, tx0, tm_x0)
        tm_y0 = jnp.where(upd, ty0, tm_y0)
        tm_x1 = jnp.where(upd, tx1, tm_x1)
        tm_y1 = jnp.where(upd, ty1, tm_y1)

    for t in range(_T):
        tx0, ty0, tx1, ty1 = truth_coords[t]
        mask = iota == bp_idx[t]
        bv = jnp.where(mask, 2.0, bv)
        tm_x0 = jnp.where(mask, tx0, tm_x0)
        tm_y0 = jnp.where(mask, ty0, tm_y0)
        tm_x1 = jnp.where(mask, tx1, tm_x1)
        tm_y1 = jnp.where(mask, ty1, tm_y1)

    pos = bv >= _THRESHOLD
    np_i = jnp.sum(pos.astype(jnp.int32))

    gx = ((tm_x0 + tm_x1) / 2.0 - pcx) / (_V0 * pw)
    gy = ((tm_y0 + tm_y1) / 2.0 - pcy) / (_V0 * ph)
    gw = jnp.log((tm_x1 - tm_x0) / pw) / _V1
    gh = jnp.log((tm_y1 - tm_y0) / ph) / _V1
    sl1 = jnp.zeros((1, P), dtype=jnp.float32)
    for l_ref, g in ((l0_ref, gx), (l1_ref, gy), (l2_ref, gw), (l3_ref, gh)):
        d = l_ref[...][0] - g
        a = jnp.abs(d)
        sl1 = sl1 + jnp.where(a < 1.0, 0.5 * d * d, a - 0.5)
    sl1_sum = jnp.sum(jnp.where(pos, sl1, 0.0))

    dm = c1_ref[...][0] - c0_ref[...][0]
    lg = jnp.log(1.0 + jnp.exp(-jnp.abs(dm)))
    ce_pos = jnp.maximum(-dm, 0.0) + lg
    v_neg = jnp.maximum(dm, 0.0) + lg
    ce_pos_sum = jnp.sum(jnp.where(pos, ce_pos, 0.0))
    v = jnp.where(pos, 0.0, v_neg)
    vb = lax.bitcast_convert_type(v, jnp.int32)

    k = jnp.minimum(_NEG_RATIO * np_i, P - 1)

    def bs_body(_, carry):
        lo, hi = carry
        mid = lo + (hi - lo) // 2
        cnt = jnp.sum((vb >= mid).astype(jnp.int32))
        good = cnt >= k
        return (jnp.where(good, mid, lo), jnp.where(good, hi, mid))

    lo, _ = lax.fori_loop(0, 31, bs_body, (jnp.int32(0), jnp.int32(0x7F800000)))
    t_bits = lo
    above = vb > t_bits
    ca = jnp.sum(above.astype(jnp.int32))
    t_val = lax.bitcast_convert_type(t_bits, jnp.float32)
    neg_sum = jnp.sum(jnp.where(above, v, 0.0)) + (k - ca).astype(jnp.float32) * t_val
    sel_cnt = np_i + jnp.where(t_bits > 0, k, ca)

    @pl.when(b == 0)
    def _init():
        acc_ref[0] = 0.0
        acc_ref[1] = 0.0
        acc_ref[2] = 0.0
        acc_ref[3] = 0.0

    acc_ref[0] = acc_ref[0] + sl1_sum
    acc_ref[1] = acc_ref[1] + ce_pos_sum + neg_sum
    acc_ref[2] = acc_ref[2] + sel_cnt.astype(jnp.float32)
    acc_ref[3] = acc_ref[3] + np_i.astype(jnp.float32)

    @pl.when(b == nb - 1)
    def _emit():
        n = acc_ref[3]
        out_ref[0] = acc_ref[0] / n
        out_ref[1] = acc_ref[1] / acc_ref[2] / n


def _run(l0, l1, l2, l3, c0, c1, pcx, pcy, pw, ph, tx0, ty0, tx1, ty1,
         interpret=False):
    B, P = l0.shape
    l0, l1, l2, l3, c0, c1 = (x.reshape(B, 1, P) for x in (l0, l1, l2, l3, c0, c1))
    row = lambda b: (b, 0, 0)
    fixed = lambda b: (0, 0)
    smem = pl.BlockSpec(memory_space=pltpu.SMEM)
    rspec = pl.BlockSpec((1, 1, P), row)
    return pl.pallas_call(
        _row_kernel,
        grid=(B,),
        in_specs=[
            pl.BlockSpec((1, P), fixed), pl.BlockSpec((1, P), fixed),
            pl.BlockSpec((1, P), fixed), pl.BlockSpec((1, P), fixed),
            smem, smem, smem, smem,
            rspec, rspec, rspec, rspec, rspec, rspec,
        ],
        out_specs=pl.BlockSpec(memory_space=pltpu.SMEM),
        out_shape=jax.ShapeDtypeStruct((2,), jnp.float32),
        scratch_shapes=[pltpu.SMEM((4,), jnp.float32)],
        interpret=interpret,
    )(pcx, pcy, pw, ph, tx0, ty0, tx1, ty1, l0, l1, l2, l3, c0, c1)


def kernel(loc_data, conf_data, priors, targets):
    B, P, _ = loc_data.shape
    l0, l1, l2, l3 = (loc_data[:, :, i] for i in range(4))
    c0 = conf_data[:, :, 0]
    c1 = conf_data[:, :, 1]
    pcx = priors[:, 0].reshape(1, P)
    pcy = priors[:, 1].reshape(1, P)
    pw = priors[:, 2].reshape(1, P)
    ph = priors[:, 3].reshape(1, P)
    tx0 = targets[:, :, 0]
    ty0 = targets[:, :, 1]
    tx1 = targets[:, :, 2]
    ty1 = targets[:, :, 3]
    out = _run(l0, l1, l2, l3, c0, c1, pcx, pcy, pw, ph, tx0, ty0, tx1, ty1)
    return out[0], out[1]


# speedup vs baseline: 19.5216x; 2.7667x over previous
import jax
import jax.numpy as jnp
from jax import lax
from jax.experimental import pallas as pl
from jax.experimental.pallas import tpu as pltpu

_THRESHOLD = 0.35
_V0, _V1 = 0.1, 0.2
_NEG_RATIO = 3
_T = 16
_BG = 8


def _match_kernel(pcx_ref, pcy_ref, pw_ref, ph_ref,
                  tx0_ref, ty0_ref, tx1_ref, ty1_ref,
                  l0_ref, l1_ref, l2_ref, l3_ref,
                  c0_ref, c1_ref,
                  v_ref, scal_ref):
    G, P = l0_ref.shape

    pcx = pcx_ref[...]
    pcy = pcy_ref[...]
    pw = pw_ref[...]
    ph = ph_ref[...]
    px0 = pcx - pw / 2.0
    py0 = pcy - ph / 2.0
    px1 = pcx + pw / 2.0
    py1 = pcy + ph / 2.0
    area_p = (px1 - px0) * (py1 - py0)

    iota = lax.broadcasted_iota(jnp.int32, (1, P), 1)

    tx0a = tx0_ref[...]
    ty0a = ty0_ref[...]
    tx1a = tx1_ref[...]
    ty1a = ty1_ref[...]

    bv = jnp.full((G, P), -1.0, dtype=jnp.float32)
    tm_x0 = jnp.zeros((G, P), dtype=jnp.float32)
    tm_y0 = jnp.zeros((G, P), dtype=jnp.float32)
    tm_x1 = jnp.zeros((G, P), dtype=jnp.float32)
    tm_y1 = jnp.zeros((G, P), dtype=jnp.float32)
    bp_idx = []

    for t in range(_T):
        tx0 = tx0a[:, t:t + 1]
        ty0 = ty0a[:, t:t + 1]
        tx1 = tx1a[:, t:t + 1]
        ty1 = ty1a[:, t:t + 1]
        iw = jnp.clip(jnp.minimum(tx1, px1) - jnp.maximum(tx0, px0), 0.0, None)
        ih = jnp.clip(jnp.minimum(ty1, py1) - jnp.maximum(ty0, py0), 0.0, None)
        inter = iw * ih
        area_t = (tx1 - tx0) * (ty1 - ty0)
        union = area_t + area_p - inter
        iou = inter / union
        m_t = jnp.max(iou, axis=1, keepdims=True)
        idx_t = jnp.min(jnp.where(iou == m_t, iota, P), axis=1, keepdims=True)
        bp_idx.append(idx_t)
        upd = iou > bv
        bv = jnp.where(upd, iou, bv)
        tm_x0 = jnp.where(upd, tx0, tm_x0)
        tm_y0 = jnp.where(upd, ty0, tm_y0)
        tm_x1 = jnp.where(upd, tx1, tm_x1)
        tm_y1 = jnp.where(upd, ty1, tm_y1)

    for t in range(_T):
        mask = iota == bp_idx[t]
        bv = jnp.where(mask, 2.0, bv)
        tm_x0 = jnp.where(mask, tx0a[:, t:t + 1], tm_x0)
        tm_y0 = jnp.where(mask, ty0a[:, t:t + 1], tm_y0)
        tm_x1 = jnp.where(mask, tx1a[:, t:t + 1], tm_x1)
        tm_y1 = jnp.where(mask, ty1a[:, t:t + 1], tm_y1)

    pos = bv >= _THRESHOLD
    np_f = jnp.sum(pos.astype(jnp.float32), axis=1, keepdims=True)

    gx = ((tm_x0 + tm_x1) / 2.0 - pcx) / (_V0 * pw)
    gy = ((tm_y0 + tm_y1) / 2.0 - pcy) / (_V0 * ph)
    gw = jnp.log((tm_x1 - tm_x0) / pw) / _V1
    gh = jnp.log((tm_y1 - tm_y0) / ph) / _V1
    sl1 = jnp.zeros((G, P), dtype=jnp.float32)
    for l_ref, g in ((l0_ref, gx), (l1_ref, gy), (l2_ref, gw), (l3_ref, gh)):
        d = l_ref[...] - g
        a = jnp.abs(d)
        sl1 = sl1 + jnp.where(a < 1.0, 0.5 * d * d, a - 0.5)
    sl1_sum = jnp.sum(jnp.where(pos, sl1, 0.0), axis=1, keepdims=True)

    dm = c1_ref[...] - c0_ref[...]
    lg = jnp.log(1.0 + jnp.exp(-jnp.abs(dm)))
    ce_pos = jnp.maximum(-dm, 0.0) + lg
    v_neg = jnp.maximum(dm, 0.0) + lg
    ce_pos_sum = jnp.sum(jnp.where(pos, ce_pos, 0.0), axis=1, keepdims=True)
    v_ref[...] = jnp.where(pos, 0.0, v_neg)
    scal_ref[...] = jnp.concatenate([np_f, sl1_sum, ce_pos_sum], axis=1)


def _mine_kernel(v_ref, scal_ref, out_ref):
    B, P = v_ref.shape
    v = v_ref[...]
    vb = lax.bitcast_convert_type(v, jnp.int32)
    scal = scal_ref[...]
    np_i = scal[:, 0:1].astype(jnp.int32)
    k = jnp.minimum(_NEG_RATIO * np_i, P - 1)

    def bs_body(_, carry):
        lo, hi = carry
        mid = lo + (hi - lo) // 2
        cnt = jnp.sum((vb >= mid).astype(jnp.int32), axis=1, keepdims=True)
        good = cnt >= k
        return (jnp.where(good, mid, lo), jnp.where(good, hi, mid))

    lo0 = jnp.zeros((B, 1), dtype=jnp.int32)
    hi0 = jnp.full((B, 1), 0x7F800000, dtype=jnp.int32)
    t_bits, _ = lax.fori_loop(0, 31, bs_body, (lo0, hi0))
    above = vb > t_bits
    ca = jnp.sum(above.astype(jnp.int32), axis=1, keepdims=True)
    t_val = lax.bitcast_convert_type(t_bits, jnp.float32)
    neg_sum = (jnp.sum(jnp.where(above, v, 0.0), axis=1, keepdims=True)
               + (k - ca).astype(jnp.float32) * t_val)
    sel_cnt = np_i + jnp.where(t_bits > 0, k, ca)

    n = jnp.sum(scal[:, 0:1])
    out_ref[0] = jnp.sum(scal[:, 1:2]) / n
    out_ref[1] = ((jnp.sum(scal[:, 2:3]) + jnp.sum(neg_sum))
                  / jnp.sum(sel_cnt).astype(jnp.float32)) / n


def _run(l0, l1, l2, l3, c0, c1, pcx, pcy, pw, ph, tx0, ty0, tx1, ty1,
         interpret=False):
    B, P = l0.shape
    row = lambda b: (b, 0)
    fixed = lambda b: (0, 0)
    rspec = pl.BlockSpec((_BG, P), row)
    tspec = pl.BlockSpec((_BG, _T), row)
    v, scal = pl.pallas_call(
        _match_kernel,
        grid=(B // _BG,),
        in_specs=[
            pl.BlockSpec((1, P), fixed), pl.BlockSpec((1, P), fixed),
            pl.BlockSpec((1, P), fixed), pl.BlockSpec((1, P), fixed),
            tspec, tspec, tspec, tspec,
            rspec, rspec, rspec, rspec, rspec, rspec,
        ],
        out_specs=[rspec, pl.BlockSpec((_BG, 3), row)],
        out_shape=[jax.ShapeDtypeStruct((B, P), jnp.float32),
                   jax.ShapeDtypeStruct((B, 3), jnp.float32)],
        interpret=interpret,
    )(pcx, pcy, pw, ph, tx0, ty0, tx1, ty1, l0, l1, l2, l3, c0, c1)

    return pl.pallas_call(
        _mine_kernel,
        in_specs=[pl.BlockSpec(None), pl.BlockSpec(None)],
        out_specs=pl.BlockSpec(memory_space=pltpu.SMEM),
        out_shape=jax.ShapeDtypeStruct((2,), jnp.float32),
        interpret=interpret,
    )(v, scal)


def kernel(loc_data, conf_data, priors, targets):
    B, P, _ = loc_data.shape
    l0, l1, l2, l3 = (loc_data[:, :, i] for i in range(4))
    c0 = conf_data[:, :, 0]
    c1 = conf_data[:, :, 1]
    pcx = priors[:, 0].reshape(1, P)
    pcy = priors[:, 1].reshape(1, P)
    pw = priors[:, 2].reshape(1, P)
    ph = priors[:, 3].reshape(1, P)
    tx0 = targets[:, :, 0]
    ty0 = targets[:, :, 1]
    tx1 = targets[:, :, 2]
    ty1 = targets[:, :, 3]
    out = _run(l0, l1, l2, l3, c0, c1, pcx, pcy, pw, ph, tx0, ty0, tx1, ty1)
    return out[0], out[1]
